# baseline (speedup 1.0000x reference)
import functools
import jax
import jax.numpy as jnp
from jax import lax
from jax.experimental import pallas as pl
from jax.experimental.pallas import tpu as pltpu
from jax.experimental.pallas import tpu_sc as plsc

NUM_CH = 192
ROW = 50176
NB = 1536
NW = 32
BPW = NB // NW


def _sc_body(x_hbm, idx_hbm, out_hbm, idx_v, sem):
    wid = lax.axis_index("s") * 2 + lax.axis_index("c")
    base = wid * BPW
    pltpu.sync_copy(idx_hbm.at[pl.ds(base, BPW)], idx_v)
    for i in range(BPW):
        src = idx_v[pl.ds((i // 16) * 16, 16)][i % 16]
        pltpu.async_copy(
            x_hbm.at[pl.ds(src, 1)], out_hbm.at[pl.ds(base + i, 1)], sem
        )
    for i in range(BPW):
        pltpu.make_async_copy(
            x_hbm.at[pl.ds(0, 1)], out_hbm.at[pl.ds(base + i, 1)], sem
        ).wait()


def kernel(x, permutation):
    b, c, h, w = x.shape
    xr = x.reshape(NB, ROW)
    idx = (
        jnp.arange(b, dtype=jnp.int32)[:, None] * c
        + permutation.astype(jnp.int32)[None, :]
    ).reshape(NB)
    mesh = plsc.VectorSubcoreMesh(core_axis_name="c", subcore_axis_name="s")
    out = pl.kernel(
        _sc_body,
        mesh=mesh,
        out_type=jax.ShapeDtypeStruct((NB, ROW), x.dtype),
        scratch_types=[
            pltpu.VMEM((BPW,), jnp.int32),
            pltpu.SemaphoreType.DMA,
        ],
    )(xr, idx)
    return out.reshape(b, c, h, w)


# speedup vs baseline: 11.2809x; 11.2809x over previous
import jax
import jax.numpy as jnp
from jax import lax
from jax.experimental import pallas as pl
from jax.experimental.pallas import tpu as pltpu
from jax.experimental.pallas import tpu_sc as plsc

ROW = 50176
NB = 1536
NW = 32
BPW = NB // NW


def _sc_body(x_hbm, idx_hbm, out_hbm, idx_v, buf0, buf1, g0, g1, s0, s1):
    wid = lax.axis_index("s") * 2 + lax.axis_index("c")
    base = wid * BPW
    pltpu.sync_copy(idx_hbm.at[pl.ds(base, BPW)], idx_v)

    bufs = (buf0, buf1)
    gsems = (g0, g1)
    ssems = (s0, s1)

    def src_row(i):
        return idx_v[pl.ds((i // 16) * 16, 16)][i % 16]

    def start_gather(i):
        pltpu.async_copy(
            x_hbm.at[pl.ds(src_row(i), 1)], bufs[i % 2], gsems[i % 2]
        )

    def wait_gather(i):
        pltpu.make_async_copy(
            x_hbm.at[pl.ds(0, 1)], bufs[i % 2], gsems[i % 2]
        ).wait()

    def start_store(i):
        pltpu.async_copy(
            bufs[i % 2], out_hbm.at[pl.ds(base + i, 1)], ssems[i % 2]
        )

    def wait_store(i):
        pltpu.make_async_copy(
            bufs[i % 2], out_hbm.at[pl.ds(base + i, 1)], ssems[i % 2]
        ).wait()

    start_gather(0)
    start_gather(1)
    for i in range(BPW):
        wait_gather(i)
        start_store(i)
        if i + 2 < BPW:
            wait_store(i)
            start_gather(i + 2)
    wait_store(BPW - 2)
    wait_store(BPW - 1)


def kernel(x, permutation):
    b, c, h, w = x.shape
    xr = x.reshape(NB, ROW)
    idx = (
        jnp.arange(b, dtype=jnp.int32)[:, None] * c
        + permutation.astype(jnp.int32)[None, :]
    ).reshape(NB)
    mesh = plsc.VectorSubcoreMesh(core_axis_name="c", subcore_axis_name="s")
    out = pl.kernel(
        _sc_body,
        mesh=mesh,
        out_type=jax.ShapeDtypeStruct((NB, ROW), x.dtype),
        scratch_types=[
            pltpu.VMEM((BPW,), jnp.int32),
            pltpu.VMEM((1, ROW), jnp.float32),
            pltpu.VMEM((1, ROW), jnp.float32),
            pltpu.SemaphoreType.DMA,
            pltpu.SemaphoreType.DMA,
            pltpu.SemaphoreType.DMA,
            pltpu.SemaphoreType.DMA,
        ],
    )(xr, idx)
    return out.reshape(b, c, h, w)
